# Initial kernel scaffold; baseline (speedup 1.0000x reference)
import jax
import jax.numpy as jnp
from jax.experimental import pallas as pl


def kernel(x, edge_index, W1, b1, W2, b2):
    raise NotImplementedError("write your pallas kernel here")


# speedup vs baseline: 10.3802x; 10.3802x over previous
import functools

import jax
import jax.numpy as jnp
from jax import lax
from jax.experimental import pallas as pl
from jax.experimental.pallas import tpu as pltpu
from jax.experimental.pallas import tpu_sc as plsc

NS = 16
NC = 2
NW = NC * NS
CH = 128


def _cdiv(a, b):
    return (a + b - 1) // b


def _make_sc_deg(n_pad, cpt, ddum):
    rpt = n_pad // NS
    mesh = plsc.VectorSubcoreMesh(
        core_axis_name="c", subcore_axis_name="s",
        num_cores=NC, num_subcores=NS)

    @functools.partial(
        pl.kernel,
        out_type=jax.ShapeDtypeStruct((NC, n_pad), jnp.float32),
        mesh=mesh,
        scratch_types=[
            pltpu.VMEM((rpt,), jnp.float32),
            pltpu.VMEM((CH,), jnp.float32),
            pltpu.VMEM((CH,), jnp.int32),
            pltpu.VMEM((CH,), jnp.int32),
            pltpu.VMEM((CH,), jnp.int32),
            pltpu.VMEM_SHARED((n_pad,), jnp.float32),
        ],
    )
    def deg_k(src_hbm, dst_hbm, deg_hbm, zb_v, ones_v, srow_v, drow_v,
              idx_v, deg_sh):
        c = lax.axis_index("c")
        s = lax.axis_index("s")
        w = c * NS + s
        zv = jnp.zeros((16,), jnp.float32)
        ov = jnp.ones((16,), jnp.float32)

        def zfill(r, carry):
            zb_v[pl.ds(r * 16, 16)] = zv
            return carry
        lax.fori_loop(0, rpt // 16, zfill, 0)

        def ofill(r, carry):
            ones_v[pl.ds(r * 16, 16)] = ov
            return carry
        lax.fori_loop(0, CH // 16, ofill, 0)
        pltpu.sync_copy(zb_v, deg_sh.at[pl.ds(s * rpt, rpt)])
        plsc.subcore_barrier()

        def chunk(j, carry):
            pltpu.sync_copy(src_hbm.at[w, j], srow_v)
            pltpu.sync_copy(dst_hbm.at[w, j], drow_v)
            for k in range(CH // 16):
                sv = srow_v[pl.ds(k * 16, 16)]
                dv = drow_v[pl.ds(k * 16, 16)]
                idx_v[pl.ds(k * 16, 16)] = jnp.where(sv == dv, ddum, dv)
            pltpu.sync_copy(ones_v, deg_sh.at[idx_v], add=True)
            return carry
        lax.fori_loop(0, cpt, chunk, 0)
        plsc.subcore_barrier()
        pltpu.sync_copy(deg_sh.at[pl.ds(s * rpt, rpt)],
                        deg_hbm.at[c, pl.ds(s * rpt, rpt)])

    return deg_k


def _make_sc_agg(n_pad, d, cpt, zrow):
    rpt = n_pad // NS
    mesh = plsc.VectorSubcoreMesh(
        core_axis_name="c", subcore_axis_name="s",
        num_cores=NC, num_subcores=NS)

    @functools.partial(
        pl.kernel,
        out_type=jax.ShapeDtypeStruct((NC, n_pad, d), jnp.float32),
        mesh=mesh,
        scratch_types=[
            pltpu.VMEM((CH,), jnp.int32),
            pltpu.VMEM((CH,), jnp.int32),
            pltpu.VMEM((CH,), jnp.int32),
            pltpu.VMEM((CH,), jnp.int32),
            pltpu.VMEM((CH, d), jnp.float32),
            pltpu.VMEM_SHARED((n_pad, d), jnp.float32),
            pltpu.SemaphoreType.DMA,
        ],
    )
    def agg_k(hs_hbm, src_hbm, dst_hbm, out_hbm, srow_v, drow_v, sidx_v,
              didx_v, buf_a, s_sh, sem_a):
        c = lax.axis_index("c")
        s = lax.axis_index("s")
        w = c * NS + s
        zv = jnp.zeros((16,), jnp.float32)

        def zrow_init(r, carry):
            for k in range(d // 16):
                buf_a[r, pl.ds(k * 16, 16)] = zv
            return carry
        lax.fori_loop(0, CH, zrow_init, 0)

        def zstripe(t, carry):
            pltpu.sync_copy(buf_a, s_sh.at[pl.ds(s * rpt + t * CH, CH)])
            return carry
        lax.fori_loop(0, rpt // CH, zstripe, 0)
        plsc.subcore_barrier()

        def chunk(j, carry):
            pltpu.sync_copy(src_hbm.at[w, j], srow_v)
            pltpu.sync_copy(dst_hbm.at[w, j], drow_v)
            for k in range(CH // 16):
                sv = srow_v[pl.ds(k * 16, 16)]
                dv = drow_v[pl.ds(k * 16, 16)]
                sidx_v[pl.ds(k * 16, 16)] = jnp.where(sv == dv, zrow, sv)
                didx_v[pl.ds(k * 16, 16)] = dv
            pltpu.async_copy(hs_hbm.at[sidx_v], buf_a, sem_a).wait()
            pltpu.sync_copy(buf_a, s_sh.at[didx_v], add=True)
            return carry
        lax.fori_loop(0, cpt, chunk, 0)
        plsc.subcore_barrier()
        pltpu.sync_copy(s_sh.at[pl.ds(s * rpt, rpt)],
                        out_hbm.at[c, pl.ds(s * rpt, rpt)])

    return agg_k


def _dinv_block(deg_ref, blk, r, n):
    dg = deg_ref[...]
    drow = dg[0] + dg[1]
    rid = lax.broadcasted_iota(jnp.int32, (r, 1), 0) + blk * r
    deg = drow + jnp.where(rid < n, 1.0, 0.0)
    return jnp.where(deg > 0, lax.rsqrt(jnp.maximum(deg, 1e-12)), 0.0)


def _tc_in(x_p, w, deg2, r, n):
    n_pad, d = x_p.shape
    g = n_pad // r

    def body(xb, wb, degb, ob):
        dinv = _dinv_block(degb, pl.program_id(0), r, n)
        h = jnp.dot(xb[...], wb[...], preferred_element_type=jnp.float32)
        ob[...] = h * dinv

    return pl.pallas_call(
        body,
        grid=(g,),
        in_specs=[
            pl.BlockSpec((r, d), lambda k: (k, 0)),
            pl.BlockSpec((d, d), lambda k: (0, 0)),
            pl.BlockSpec((NC, r, 1), lambda k: (0, k, 0)),
        ],
        out_specs=pl.BlockSpec((r, d), lambda k: (k, 0)),
        out_shape=jax.ShapeDtypeStruct((n_pad, d), jnp.float32),
    )(x_p, w, deg2)


def _tc_mid(s2, hs, deg2, b, w, r, n):
    n_pad, d = hs.shape
    g = n_pad // r

    def body(sb, hb, degb, bb, wb, ob):
        dinv = _dinv_block(degb, pl.program_id(0), r, n)
        z = jnp.maximum(dinv * (sb[0] + sb[1] + hb[...]) + bb[...], 0.0)
        ob[...] = jnp.dot(z, wb[...], preferred_element_type=jnp.float32) * dinv

    return pl.pallas_call(
        body,
        grid=(g,),
        in_specs=[
            pl.BlockSpec((NC, r, d), lambda k: (0, k, 0)),
            pl.BlockSpec((r, d), lambda k: (k, 0)),
            pl.BlockSpec((NC, r, 1), lambda k: (0, k, 0)),
            pl.BlockSpec((1, d), lambda k: (0, 0)),
            pl.BlockSpec((d, d), lambda k: (0, 0)),
        ],
        out_specs=pl.BlockSpec((r, d), lambda k: (k, 0)),
        out_shape=jax.ShapeDtypeStruct((n_pad, d), jnp.float32),
    )(s2, hs, deg2, b, w)


def _tc_out(s2, hs, deg2, b, r, n):
    n_pad, d = hs.shape
    g = n_pad // r

    def body(sb, hb, degb, bb, ob):
        dinv = _dinv_block(degb, pl.program_id(0), r, n)
        ob[...] = dinv * (sb[0] + sb[1] + hb[...]) + bb[...]

    return pl.pallas_call(
        body,
        grid=(g,),
        in_specs=[
            pl.BlockSpec((NC, r, d), lambda k: (0, k, 0)),
            pl.BlockSpec((r, d), lambda k: (k, 0)),
            pl.BlockSpec((NC, r, 1), lambda k: (0, k, 0)),
            pl.BlockSpec((1, d), lambda k: (0, 0)),
        ],
        out_specs=pl.BlockSpec((r, d), lambda k: (k, 0)),
        out_shape=jax.ShapeDtypeStruct((n_pad, d), jnp.float32),
    )(s2, hs, deg2, b)


def kernel(x, edge_index, W1, b1, W2, b2):
    n, d = x.shape
    e = edge_index.shape[1]
    r = 1024
    n_pad = _cdiv(n, r) * r
    e_pad = _cdiv(e, NW * CH) * NW * CH
    cpt = e_pad // (NW * CH)
    zrow = n
    ddum = n + 1

    src = edge_index[0].astype(jnp.int32)
    dst = edge_index[1].astype(jnp.int32)
    src_p = jnp.pad(src, (0, e_pad - e)).reshape(NW, cpt, CH)
    dst_p = jnp.pad(dst, (0, e_pad - e)).reshape(NW, cpt, CH)
    x_p = jnp.pad(x, ((0, n_pad - n), (0, 0)))
    b1r = b1.reshape(1, d)
    b2r = b2.reshape(1, d)

    deg2 = _make_sc_deg(n_pad, cpt, ddum)(src_p, dst_p).reshape(NC, n_pad, 1)
    agg = _make_sc_agg(n_pad, d, cpt, zrow)

    hs1 = _tc_in(x_p, W1, deg2, r, n)
    s1 = agg(hs1, src_p, dst_p)
    hs2 = _tc_mid(s1, hs1, deg2, b1r, W2, r, n)
    s2 = agg(hs2, src_p, dst_p)
    out = _tc_out(s2, hs2, deg2, b2r, r, n)
    return out[:n]


# speedup vs baseline: 12.9518x; 1.2477x over previous
import functools

import jax
import jax.numpy as jnp
from jax import lax
from jax.experimental import pallas as pl
from jax.experimental.pallas import tpu as pltpu
from jax.experimental.pallas import tpu_sc as plsc

NS = 16
NC = 2
NW = NC * NS
CH = 128


def _cdiv(a, b):
    return (a + b - 1) // b


def _make_sc_deg(n_pad, cpt, ddum):
    rpt = n_pad // NS
    mesh = plsc.VectorSubcoreMesh(
        core_axis_name="c", subcore_axis_name="s",
        num_cores=NC, num_subcores=NS)

    @functools.partial(
        pl.kernel,
        out_type=jax.ShapeDtypeStruct((NC, n_pad), jnp.float32),
        mesh=mesh,
        scratch_types=[
            pltpu.VMEM((rpt,), jnp.float32),
            pltpu.VMEM((CH,), jnp.float32),
            pltpu.VMEM((CH,), jnp.int32),
            pltpu.VMEM((CH,), jnp.int32),
            pltpu.VMEM((CH,), jnp.int32),
            pltpu.VMEM_SHARED((n_pad,), jnp.float32),
        ],
    )
    def deg_k(src_hbm, dst_hbm, deg_hbm, zb_v, ones_v, srow_v, drow_v,
              idx_v, deg_sh):
        c = lax.axis_index("c")
        s = lax.axis_index("s")
        w = c * NS + s
        zv = jnp.zeros((16,), jnp.float32)
        ov = jnp.ones((16,), jnp.float32)

        def zfill(r, carry):
            zb_v[pl.ds(r * 16, 16)] = zv
            return carry
        lax.fori_loop(0, rpt // 16, zfill, 0)

        def ofill(r, carry):
            ones_v[pl.ds(r * 16, 16)] = ov
            return carry
        lax.fori_loop(0, CH // 16, ofill, 0)
        pltpu.sync_copy(zb_v, deg_sh.at[pl.ds(s * rpt, rpt)])
        plsc.subcore_barrier()

        def chunk(j, carry):
            pltpu.sync_copy(src_hbm.at[w, j], srow_v)
            pltpu.sync_copy(dst_hbm.at[w, j], drow_v)
            for k in range(CH // 16):
                sv = srow_v[pl.ds(k * 16, 16)]
                dv = drow_v[pl.ds(k * 16, 16)]
                idx_v[pl.ds(k * 16, 16)] = jnp.where(sv == dv, ddum, dv)
            pltpu.sync_copy(ones_v, deg_sh.at[idx_v], add=True)
            return carry
        lax.fori_loop(0, cpt, chunk, 0)
        plsc.subcore_barrier()
        pltpu.sync_copy(deg_sh.at[pl.ds(s * rpt, rpt)],
                        deg_hbm.at[c, pl.ds(s * rpt, rpt)])

    return deg_k


def _make_sc_agg(n_pad, d, cpt, zrow):
    rpt = n_pad // NS
    mesh = plsc.VectorSubcoreMesh(
        core_axis_name="c", subcore_axis_name="s",
        num_cores=NC, num_subcores=NS)

    @functools.partial(
        pl.kernel,
        out_type=jax.ShapeDtypeStruct((NC, n_pad, d), jnp.float32),
        mesh=mesh,
        scratch_types=[
            pltpu.VMEM((cpt, CH), jnp.int32),
            pltpu.VMEM((cpt, CH), jnp.int32),
            pltpu.VMEM((CH,), jnp.int32),
            pltpu.VMEM((CH,), jnp.int32),
            pltpu.VMEM((CH, d), jnp.float32),
            pltpu.VMEM_SHARED((n_pad, d), jnp.float32),
            pltpu.SemaphoreType.DMA,
        ],
    )
    def agg_k(hs_hbm, src_hbm, dst_hbm, out_hbm, src_v, dst_v, sidx_v,
              didx_v, buf_a, s_sh, sem_a):
        c = lax.axis_index("c")
        s = lax.axis_index("s")
        w = c * NS + s
        pltpu.sync_copy(src_hbm.at[w], src_v)
        pltpu.sync_copy(dst_hbm.at[w], dst_v)
        zv = jnp.zeros((16,), jnp.float32)

        def zrow_init(r, carry):
            for k in range(d // 16):
                buf_a[r, pl.ds(k * 16, 16)] = zv
            return carry
        lax.fori_loop(0, CH, zrow_init, 0)

        def zstripe(t, carry):
            pltpu.sync_copy(buf_a, s_sh.at[pl.ds(s * rpt + t * CH, CH)])
            return carry
        lax.fori_loop(0, rpt // CH, zstripe, 0)
        plsc.subcore_barrier()

        def chunk(j, carry):
            for k in range(CH // 16):
                sv = src_v[j, pl.ds(k * 16, 16)]
                dv = dst_v[j, pl.ds(k * 16, 16)]
                sidx_v[pl.ds(k * 16, 16)] = jnp.where(sv == dv, zrow, sv)
                didx_v[pl.ds(k * 16, 16)] = dv
            pltpu.async_copy(hs_hbm.at[sidx_v], buf_a, sem_a).wait()
            pltpu.sync_copy(buf_a, s_sh.at[didx_v], add=True)
            return carry
        lax.fori_loop(0, cpt, chunk, 0)
        plsc.subcore_barrier()
        pltpu.sync_copy(s_sh.at[pl.ds(s * rpt, rpt)],
                        out_hbm.at[c, pl.ds(s * rpt, rpt)])

    return agg_k


def _dinv_block(deg_ref, blk, r, n):
    dg = deg_ref[...]
    drow = dg[0] + dg[1]
    rid = lax.broadcasted_iota(jnp.int32, (r, 1), 0) + blk * r
    deg = drow + jnp.where(rid < n, 1.0, 0.0)
    return jnp.where(deg > 0, lax.rsqrt(jnp.maximum(deg, 1e-12)), 0.0)


def _tc_mm(x_p, w, r):
    n_pad, d = x_p.shape
    g = n_pad // r

    def body(xb, wb, ob):
        ob[...] = jnp.dot(xb[...], wb[...], preferred_element_type=jnp.float32)

    return pl.pallas_call(
        body,
        grid=(g,),
        in_specs=[
            pl.BlockSpec((r, d), lambda k: (k, 0)),
            pl.BlockSpec((d, d), lambda k: (0, 0)),
        ],
        out_specs=pl.BlockSpec((r, d), lambda k: (k, 0)),
        out_shape=jax.ShapeDtypeStruct((n_pad, d), jnp.float32),
    )(x_p, w)


def _tc_scale(h, deg2, r, n):
    n_pad, d = h.shape
    g = n_pad // r

    def body(hb, degb, ob):
        dinv = _dinv_block(degb, pl.program_id(0), r, n)
        ob[...] = hb[...] * dinv

    return pl.pallas_call(
        body,
        grid=(g,),
        in_specs=[
            pl.BlockSpec((r, d), lambda k: (k, 0)),
            pl.BlockSpec((NC, r, 1), lambda k: (0, k, 0)),
        ],
        out_specs=pl.BlockSpec((r, d), lambda k: (k, 0)),
        out_shape=jax.ShapeDtypeStruct((n_pad, d), jnp.float32),
    )(h, deg2)


def _tc_mid(s2, hs, deg2, b, w, r, n):
    n_pad, d = hs.shape
    g = n_pad // r

    def body(sb, hb, degb, bb, wb, ob):
        dinv = _dinv_block(degb, pl.program_id(0), r, n)
        z = jnp.maximum(dinv * (sb[0] + sb[1] + hb[...]) + bb[...], 0.0)
        ob[...] = jnp.dot(z, wb[...], preferred_element_type=jnp.float32) * dinv

    return pl.pallas_call(
        body,
        grid=(g,),
        in_specs=[
            pl.BlockSpec((NC, r, d), lambda k: (0, k, 0)),
            pl.BlockSpec((r, d), lambda k: (k, 0)),
            pl.BlockSpec((NC, r, 1), lambda k: (0, k, 0)),
            pl.BlockSpec((1, d), lambda k: (0, 0)),
            pl.BlockSpec((d, d), lambda k: (0, 0)),
        ],
        out_specs=pl.BlockSpec((r, d), lambda k: (k, 0)),
        out_shape=jax.ShapeDtypeStruct((n_pad, d), jnp.float32),
    )(s2, hs, deg2, b, w)


def _tc_out(s2, hs, deg2, b, r, n):
    n_pad, d = hs.shape
    g = n_pad // r

    def body(sb, hb, degb, bb, ob):
        dinv = _dinv_block(degb, pl.program_id(0), r, n)
        ob[...] = dinv * (sb[0] + sb[1] + hb[...]) + bb[...]

    return pl.pallas_call(
        body,
        grid=(g,),
        in_specs=[
            pl.BlockSpec((NC, r, d), lambda k: (0, k, 0)),
            pl.BlockSpec((r, d), lambda k: (k, 0)),
            pl.BlockSpec((NC, r, 1), lambda k: (0, k, 0)),
            pl.BlockSpec((1, d), lambda k: (0, 0)),
        ],
        out_specs=pl.BlockSpec((r, d), lambda k: (k, 0)),
        out_shape=jax.ShapeDtypeStruct((n_pad, d), jnp.float32),
    )(s2, hs, deg2, b)


def kernel(x, edge_index, W1, b1, W2, b2):
    n, d = x.shape
    e = edge_index.shape[1]
    r = 1024
    n_pad = _cdiv(n, r) * r
    e_pad = _cdiv(e, NW * CH) * NW * CH
    cpt = e_pad // (NW * CH)
    zrow = n
    ddum = n + 1

    src = edge_index[0].astype(jnp.int32)
    dst = edge_index[1].astype(jnp.int32)
    src_p = jnp.pad(src, (0, e_pad - e)).reshape(NW, cpt, CH)
    dst_p = jnp.pad(dst, (0, e_pad - e)).reshape(NW, cpt, CH)
    x_p = jnp.pad(x, ((0, n_pad - n), (0, 0)))
    b1r = b1.reshape(1, d)
    b2r = b2.reshape(1, d)

    h1 = _tc_mm(x_p, W1, r)
    deg2 = _make_sc_deg(n_pad, cpt, ddum)(src_p, dst_p).reshape(NC, n_pad, 1)
    agg = _make_sc_agg(n_pad, d, cpt, zrow)

    hs1 = _tc_scale(h1, deg2, r, n)
    s1 = agg(hs1, src_p, dst_p)
    hs2 = _tc_mid(s1, hs1, deg2, b1r, W2, r, n)
    s2 = agg(hs2, src_p, dst_p)
    out = _tc_out(s2, hs2, deg2, b2r, r, n)
    return out[:n]


# speedup vs baseline: 13.4333x; 1.0372x over previous
import functools

import jax
import jax.numpy as jnp
from jax import lax
from jax.experimental import pallas as pl
from jax.experimental.pallas import tpu as pltpu
from jax.experimental.pallas import tpu_sc as plsc

NS = 16
NC = 2
NW = NC * NS
CH = 128


def _cdiv(a, b):
    return (a + b - 1) // b


def _make_sc_deg(n_pad, cpt, ddum):
    rpt = n_pad // NS
    mesh = plsc.VectorSubcoreMesh(
        core_axis_name="c", subcore_axis_name="s",
        num_cores=NC, num_subcores=NS)

    @functools.partial(
        pl.kernel,
        out_type=jax.ShapeDtypeStruct((NC, n_pad), jnp.float32),
        mesh=mesh,
        scratch_types=[
            pltpu.VMEM((rpt,), jnp.float32),
            pltpu.VMEM((CH,), jnp.float32),
            pltpu.VMEM((CH,), jnp.int32),
            pltpu.VMEM((CH,), jnp.int32),
            pltpu.VMEM((CH,), jnp.int32),
            pltpu.VMEM_SHARED((n_pad,), jnp.float32),
        ],
    )
    def deg_k(src_hbm, dst_hbm, deg_hbm, zb_v, ones_v, srow_v, drow_v,
              idx_v, deg_sh):
        c = lax.axis_index("c")
        s = lax.axis_index("s")
        w = c * NS + s
        zv = jnp.zeros((16,), jnp.float32)
        ov = jnp.ones((16,), jnp.float32)

        def zfill(r, carry):
            zb_v[pl.ds(r * 16, 16)] = zv
            return carry
        lax.fori_loop(0, rpt // 16, zfill, 0)

        def ofill(r, carry):
            ones_v[pl.ds(r * 16, 16)] = ov
            return carry
        lax.fori_loop(0, CH // 16, ofill, 0)
        pltpu.sync_copy(zb_v, deg_sh.at[pl.ds(s * rpt, rpt)])
        plsc.subcore_barrier()

        def chunk(j, carry):
            pltpu.sync_copy(src_hbm.at[w, j], srow_v)
            pltpu.sync_copy(dst_hbm.at[w, j], drow_v)
            for k in range(CH // 16):
                sv = srow_v[pl.ds(k * 16, 16)]
                dv = drow_v[pl.ds(k * 16, 16)]
                idx_v[pl.ds(k * 16, 16)] = jnp.where(sv == dv, ddum, dv)
            pltpu.sync_copy(ones_v, deg_sh.at[idx_v], add=True)
            return carry
        lax.fori_loop(0, cpt, chunk, 0)
        plsc.subcore_barrier()
        pltpu.sync_copy(deg_sh.at[pl.ds(s * rpt, rpt)],
                        deg_hbm.at[c, pl.ds(s * rpt, rpt)])

    return deg_k


def _make_sc_agg(n_pad, d, cpt, zrow):
    rpt = n_pad // NS
    mesh = plsc.VectorSubcoreMesh(
        core_axis_name="c", subcore_axis_name="s",
        num_cores=NC, num_subcores=NS)

    @functools.partial(
        pl.kernel,
        out_type=jax.ShapeDtypeStruct((NC, n_pad, d), jnp.float32),
        mesh=mesh,
        scratch_types=[
            pltpu.VMEM((cpt, CH), jnp.int32),
            pltpu.VMEM((CH,), jnp.int32),
            pltpu.VMEM((2, CH), jnp.int32),
            pltpu.VMEM((CH,), jnp.int32),
            pltpu.VMEM((2, CH, d), jnp.float32),
            pltpu.VMEM_SHARED((n_pad, d), jnp.float32),
            pltpu.SemaphoreType.DMA,
        ],
    )
    def agg_k(hs_hbm, src_hbm, dst_hbm, out_hbm, dst_v, srow_v, sidx3,
              didx_v, buf3, s_sh, sem_a):
        c = lax.axis_index("c")
        s = lax.axis_index("s")
        w = c * NS + s
        pltpu.sync_copy(dst_hbm.at[w], dst_v)
        zv = jnp.zeros((16,), jnp.float32)

        def zrow_init(r, carry):
            for k in range(d // 16):
                buf3[0, r, pl.ds(k * 16, 16)] = zv
            return carry
        lax.fori_loop(0, CH, zrow_init, 0)

        def zstripe(t, carry):
            pltpu.sync_copy(buf3.at[0], s_sh.at[pl.ds(s * rpt + t * CH, CH)])
            return carry
        lax.fori_loop(0, rpt // CH, zstripe, 0)
        plsc.subcore_barrier()

        def stage_build(j, q):
            pltpu.sync_copy(src_hbm.at[w, j], srow_v)
            for k in range(CH // 16):
                sv = srow_v[pl.ds(k * 16, 16)]
                dv = dst_v[j, pl.ds(k * 16, 16)]
                sidx3[q, pl.ds(k * 16, 16)] = jnp.where(sv == dv, zrow, sv)

        def gstart(q):
            pltpu.make_async_copy(hs_hbm.at[sidx3.at[q]], buf3.at[q],
                                  sem_a).start()

        def gwait(q):
            pltpu.make_async_copy(hs_hbm.at[sidx3.at[q]], buf3.at[q],
                                  sem_a).wait()

        stage_build(0, 0)
        gstart(0)

        def chunk(j, carry):
            p = lax.rem(j, 2)
            gwait(p)

            @pl.when(j < cpt - 1)
            def _():
                stage_build(j + 1, 1 - p)
                gstart(1 - p)

            for k in range(CH // 16):
                didx_v[pl.ds(k * 16, 16)] = dst_v[j, pl.ds(k * 16, 16)]
            pltpu.sync_copy(buf3.at[p], s_sh.at[didx_v], add=True)
            return carry
        lax.fori_loop(0, cpt, chunk, 0)
        plsc.subcore_barrier()
        pltpu.sync_copy(s_sh.at[pl.ds(s * rpt, rpt)],
                        out_hbm.at[c, pl.ds(s * rpt, rpt)])

    return agg_k


def _dinv_block(deg_ref, blk, r, n):
    dg = deg_ref[...]
    drow = dg[0] + dg[1]
    rid = lax.broadcasted_iota(jnp.int32, (r, 1), 0) + blk * r
    deg = drow + jnp.where(rid < n, 1.0, 0.0)
    return jnp.where(deg > 0, lax.rsqrt(jnp.maximum(deg, 1e-12)), 0.0)


def _tc_mm(x_p, w, r):
    n_pad, d = x_p.shape
    g = n_pad // r

    def body(xb, wb, ob):
        ob[...] = jnp.dot(xb[...], wb[...], preferred_element_type=jnp.float32)

    return pl.pallas_call(
        body,
        grid=(g,),
        in_specs=[
            pl.BlockSpec((r, d), lambda k: (k, 0)),
            pl.BlockSpec((d, d), lambda k: (0, 0)),
        ],
        out_specs=pl.BlockSpec((r, d), lambda k: (k, 0)),
        out_shape=jax.ShapeDtypeStruct((n_pad, d), jnp.float32),
    )(x_p, w)


def _tc_scale(h, deg2, r, n):
    n_pad, d = h.shape
    g = n_pad // r

    def body(hb, degb, ob):
        dinv = _dinv_block(degb, pl.program_id(0), r, n)
        ob[...] = hb[...] * dinv

    return pl.pallas_call(
        body,
        grid=(g,),
        in_specs=[
            pl.BlockSpec((r, d), lambda k: (k, 0)),
            pl.BlockSpec((NC, r, 1), lambda k: (0, k, 0)),
        ],
        out_specs=pl.BlockSpec((r, d), lambda k: (k, 0)),
        out_shape=jax.ShapeDtypeStruct((n_pad, d), jnp.float32),
    )(h, deg2)


def _tc_mid(s2, hs, deg2, b, w, r, n):
    n_pad, d = hs.shape
    g = n_pad // r

    def body(sb, hb, degb, bb, wb, ob):
        dinv = _dinv_block(degb, pl.program_id(0), r, n)
        z = jnp.maximum(dinv * (sb[0] + sb[1] + hb[...]) + bb[...], 0.0)
        ob[...] = jnp.dot(z, wb[...], preferred_element_type=jnp.float32) * dinv

    return pl.pallas_call(
        body,
        grid=(g,),
        in_specs=[
            pl.BlockSpec((NC, r, d), lambda k: (0, k, 0)),
            pl.BlockSpec((r, d), lambda k: (k, 0)),
            pl.BlockSpec((NC, r, 1), lambda k: (0, k, 0)),
            pl.BlockSpec((1, d), lambda k: (0, 0)),
            pl.BlockSpec((d, d), lambda k: (0, 0)),
        ],
        out_specs=pl.BlockSpec((r, d), lambda k: (k, 0)),
        out_shape=jax.ShapeDtypeStruct((n_pad, d), jnp.float32),
    )(s2, hs, deg2, b, w)


def _tc_out(s2, hs, deg2, b, r, n):
    n_pad, d = hs.shape
    g = n_pad // r

    def body(sb, hb, degb, bb, ob):
        dinv = _dinv_block(degb, pl.program_id(0), r, n)
        ob[...] = dinv * (sb[0] + sb[1] + hb[...]) + bb[...]

    return pl.pallas_call(
        body,
        grid=(g,),
        in_specs=[
            pl.BlockSpec((NC, r, d), lambda k: (0, k, 0)),
            pl.BlockSpec((r, d), lambda k: (k, 0)),
            pl.BlockSpec((NC, r, 1), lambda k: (0, k, 0)),
            pl.BlockSpec((1, d), lambda k: (0, 0)),
        ],
        out_specs=pl.BlockSpec((r, d), lambda k: (k, 0)),
        out_shape=jax.ShapeDtypeStruct((n_pad, d), jnp.float32),
    )(s2, hs, deg2, b)


def kernel(x, edge_index, W1, b1, W2, b2):
    n, d = x.shape
    e = edge_index.shape[1]
    r = 1024
    n_pad = _cdiv(n, r) * r
    e_pad = _cdiv(e, NW * CH) * NW * CH
    cpt = e_pad // (NW * CH)
    zrow = n
    ddum = n + 1

    src = edge_index[0].astype(jnp.int32)
    dst = edge_index[1].astype(jnp.int32)
    src_p = jnp.pad(src, (0, e_pad - e)).reshape(NW, cpt, CH)
    dst_p = jnp.pad(dst, (0, e_pad - e)).reshape(NW, cpt, CH)
    x_p = jnp.pad(x, ((0, n_pad - n), (0, 0)))
    b1r = b1.reshape(1, d)
    b2r = b2.reshape(1, d)

    h1 = _tc_mm(x_p, W1, r)
    deg2 = _make_sc_deg(n_pad, cpt, ddum)(src_p, dst_p).reshape(NC, n_pad, 1)
    agg = _make_sc_agg(n_pad, d, cpt, zrow)

    hs1 = _tc_scale(h1, deg2, r, n)
    s1 = agg(hs1, src_p, dst_p)
    hs2 = _tc_mid(s1, hs1, deg2, b1r, W2, r, n)
    s2 = agg(hs2, src_p, dst_p)
    out = _tc_out(s2, hs2, deg2, b2r, r, n)
    return out[:n]
